# Initial kernel scaffold; baseline (speedup 1.0000x reference)
import jax
import jax.numpy as jnp
from jax.experimental import pallas as pl


def kernel(idx, input_pos, tok_emb, wqkv, wo, attn_norm_w, ffn_norm_w, gate_w, w1, w2, w3, final_norm_w, out_w):
    raise NotImplementedError("write your pallas kernel here")


# speedup vs baseline: 1.4286x; 1.4286x over previous
import functools

import numpy as np
import jax
import jax.numpy as jnp
from jax import lax
from jax.experimental import pallas as pl
from jax.experimental.pallas import tpu as pltpu, tpu_sc as plsc

B = 1; S = 2048; DIM = 1024; NH = 16; NLH = 8; HD = 64; IS = 2816
NE = 8; TOPK = 2; VOCAB = 32000; EPS = 1e-05; ROPE_BASE = 10000.0
QKV_D = (NH + 2 * NLH) * HD
HHD = HD // 2

F32 = jnp.float32
BF16 = jnp.bfloat16


def _bdot(a, b, dims):
    return lax.dot_general(a.astype(BF16), b.astype(BF16), dims,
                           preferred_element_type=F32)


def _freq_tables():
    freqs = 1.0 / (ROPE_BASE ** (np.arange(0, HD, 2)[: HD // 2].astype(np.float64) / HD))
    t = np.arange(S)
    fr = np.outer(t, freqs)
    return np.cos(fr).astype(np.float32), np.sin(fr).astype(np.float32)

_COS_NP, _SIN_NP = _freq_tables()

def _qkv_perm():
    perm = np.arange(QKV_D)
    for h in range(NH + NLH):
        base = h * HD
        perm[base:base + HHD] = base + 2 * np.arange(HHD)
        perm[base + HHD:base + HD] = base + 2 * np.arange(HHD) + 1
    return perm

_PERM_NP = _qkv_perm()


_SC_NC, _SC_NS = 2, 16
_SC_NW = _SC_NC * _SC_NS
_EMB_BPW = S // _SC_NW

@functools.lru_cache(maxsize=None)
def _embed_gather_kernel():
    mesh = plsc.VectorSubcoreMesh(core_axis_name="c", subcore_axis_name="s")

    @functools.partial(
        pl.kernel, mesh=mesh,
        out_type=jax.ShapeDtypeStruct((S, DIM), F32),
        scratch_types=[
            pltpu.VMEM((_EMB_BPW,), jnp.int32),
            pltpu.VMEM((_EMB_BPW, DIM), F32),
            pltpu.SemaphoreType.DMA,
        ],
    )
    def k(table_hbm, idx_hbm, out_hbm, idx_v, rows_v, sem):
        wid = lax.axis_index("s") * _SC_NC + lax.axis_index("c")
        base = wid * _EMB_BPW
        pltpu.sync_copy(idx_hbm.at[pl.ds(base, _EMB_BPW)], idx_v)
        pltpu.async_copy(table_hbm.at[idx_v], rows_v, sem).wait()
        pltpu.sync_copy(rows_v, out_hbm.at[pl.ds(base, _EMB_BPW)])

    return k


def _embed_gather(table, idx_flat):
    return _embed_gather_kernel()(table, idx_flat)


def _rowsum_fold(p):
    acc = p[:, :128]
    for c in range(1, p.shape[1] // 128):
        acc = acc + p[:, c * 128:(c + 1) * 128]
    w = 128
    while w > 1:
        acc = acc[:, :w // 2] + acc[:, w // 2:w]
        w //= 2
    return acc


def _rms(x, w):
    mean = _rowsum_fold(x * x) * jnp.float32(1.0 / x.shape[-1])
    return (x * lax.rsqrt(mean + EPS)) * w


def _qkv_body(x_ref, nw_ref, w_ref, cos_ref, sin_ref, q_ref, k_ref, v_ref):
    xn = _rms(x_ref[...], nw_ref[...])
    qkv = _bdot(xn, w_ref[...], (((1,), (1,)), ((), ())))
    cos = cos_ref[...]
    sin = sin_ref[...]

    def rope(sl):
        a = sl[:, :HHD]
        b = sl[:, HHD:]
        return jnp.concatenate([a * cos - b * sin, b * cos + a * sin], axis=1)

    q_ref[...] = jnp.stack(
        [rope(qkv[:, h * HD:(h + 1) * HD]) for h in range(NH)], axis=0)
    k_ref[...] = jnp.stack(
        [rope(qkv[:, (NH + h) * HD:(NH + h + 1) * HD]) for h in range(NLH)],
        axis=0)
    v_ref[...] = jnp.stack(
        [qkv[:, (NH + NLH + h) * HD:(NH + NLH + h + 1) * HD]
         for h in range(NLH)], axis=0)


def _qkv_call(x, nw, wp, cos, sin):
    TB = 256
    return pl.pallas_call(
        _qkv_body,
        grid=(S // TB,),
        in_specs=[
            pl.BlockSpec((TB, DIM), lambda t: (t, 0)),
            pl.BlockSpec((1, DIM), lambda t: (0, 0)),
            pl.BlockSpec((QKV_D, DIM), lambda t: (0, 0)),
            pl.BlockSpec((TB, HHD), lambda t: (t, 0)),
            pl.BlockSpec((TB, HHD), lambda t: (t, 0)),
        ],
        out_specs=[
            pl.BlockSpec((NH, TB, HD), lambda t: (0, t, 0)),
            pl.BlockSpec((NLH, TB, HD), lambda t: (0, t, 0)),
            pl.BlockSpec((NLH, TB, HD), lambda t: (0, t, 0)),
        ],
        out_shape=[
            jax.ShapeDtypeStruct((NH, S, HD), F32),
            jax.ShapeDtypeStruct((NLH, S, HD), F32),
            jax.ShapeDtypeStruct((NLH, S, HD), F32),
        ],
        compiler_params=pltpu.CompilerParams(
            dimension_semantics=("parallel",)),
    )(x, nw, wp, cos, sin)


def _attn_body(q_ref, k_ref, v_ref, o_ref):
    qb = pl.program_id(1)
    q = q_ref[0]
    k = k_ref[0]
    BQ = q.shape[0]
    s = _bdot(q, k, (((1,), (1,)), ((), ()))) * (1.0 / np.sqrt(HD))
    row = lax.broadcasted_iota(jnp.int32, s.shape, 0) + qb * BQ
    col = lax.broadcasted_iota(jnp.int32, s.shape, 1)
    s = jnp.where(col <= row, s, jnp.float32(-1e30))
    m = jnp.max(s, axis=1, keepdims=True)
    p = jnp.exp(s - m)
    p = p * pl.reciprocal(_rowsum_fold(p), approx=False)
    o_ref[0] = _bdot(p, v_ref[0], (((1,), (0,)), ((), ())))


def _attn_call(q3, k3, v3):
    BQ = 256
    return pl.pallas_call(
        _attn_body,
        grid=(NH, S // BQ),
        in_specs=[
            pl.BlockSpec((1, BQ, HD), lambda h, t: (h, t, 0)),
            pl.BlockSpec((1, S, HD), lambda h, t: (h // 2, 0, 0)),
            pl.BlockSpec((1, S, HD), lambda h, t: (h // 2, 0, 0)),
        ],
        out_specs=pl.BlockSpec((1, BQ, HD), lambda h, t: (h, t, 0)),
        out_shape=jax.ShapeDtypeStruct((NH, S, HD), F32),
        compiler_params=pltpu.CompilerParams(
            dimension_semantics=("arbitrary", "parallel")),
    )(q3, k3, v3)


def _wo_body(x_ref, a_ref, wo_ref, nw_ref, h_ref, hn_ref):
    a3 = a_ref[...]
    a = jnp.concatenate([a3[i] for i in range(NH)], axis=1)
    h = x_ref[...] + _bdot(a, wo_ref[...], (((1,), (1,)), ((), ())))
    h_ref[...] = h
    hn_ref[...] = _rms(h, nw_ref[...])


def _wo_call(x, attn, wo, nw):
    TB = 256
    return pl.pallas_call(
        _wo_body,
        grid=(S // TB,),
        in_specs=[
            pl.BlockSpec((TB, DIM), lambda t: (t, 0)),
            pl.BlockSpec((NH, TB, HD), lambda t: (0, t, 0)),
            pl.BlockSpec((DIM, DIM), lambda t: (0, 0)),
            pl.BlockSpec((1, DIM), lambda t: (0, 0)),
        ],
        out_specs=[
            pl.BlockSpec((TB, DIM), lambda t: (t, 0)),
            pl.BlockSpec((TB, DIM), lambda t: (t, 0)),
        ],
        out_shape=[
            jax.ShapeDtypeStruct((S, DIM), F32),
            jax.ShapeDtypeStruct((S, DIM), F32),
        ],
        compiler_params=pltpu.CompilerParams(
            dimension_semantics=("parallel",)),
    )(x, attn, wo, nw)


def _gate_body(xn_ref, gw_ref, we_ref):
    logits = _bdot(xn_ref[...], gw_ref[...], (((1,), (1,)), ((), ())))
    m = jnp.max(logits, axis=1, keepdims=True)
    e = jnp.exp(logits - m)
    iot = lax.broadcasted_iota(jnp.int32, e.shape, 1)
    i1 = jnp.min(jnp.where(e == 1.0, iot, NE), axis=1, keepdims=True)
    e2 = jnp.where(iot == i1, -1.0, e)
    m2 = jnp.max(e2, axis=1, keepdims=True)
    i2 = jnp.min(jnp.where(e2 == m2, iot, NE), axis=1, keepdims=True)
    denom = 1.0 + m2
    we_ref[...] = (jnp.where(iot == i1, 1.0, 0.0)
                   + jnp.where(iot == i2, m2, 0.0)) / denom


def _gate_call(hn, gate_w):
    TB = 256
    return pl.pallas_call(
        _gate_body,
        grid=(S // TB,),
        in_specs=[
            pl.BlockSpec((TB, DIM), lambda t: (t, 0)),
            pl.BlockSpec((NE, DIM), lambda t: (0, 0)),
        ],
        out_specs=pl.BlockSpec((TB, NE), lambda t: (t, 0)),
        out_shape=jax.ShapeDtypeStruct((S, NE), F32),
        compiler_params=pltpu.CompilerParams(
            dimension_semantics=("parallel",)),
    )(hn, gate_w)


def _moe_body(xn_ref, w1_ref, w3_ref, w2_ref, we_ref, h_ref, o_ref):
    e = pl.program_id(1)
    i = pl.program_id(2)

    @pl.when((e == 0) & (i == 0))
    def _init():
        o_ref[...] = h_ref[...]

    xn = xn_ref[...]
    h1 = _bdot(xn, w1_ref[0], (((1,), (1,)), ((), ())))
    h3 = _bdot(xn, w3_ref[0], (((1,), (1,)), ((), ())))
    g = h1 * jax.nn.sigmoid(h1) * h3
    y = _bdot(g, w2_ref[0], (((1,), (1,)), ((), ())))
    iot = lax.broadcasted_iota(jnp.int32, we_ref.shape, 1)
    wcol = jnp.sum(jnp.where(iot == e, we_ref[...], 0.0), axis=1)
    o_ref[...] += y * wcol[:, None]


def _moe_call(hn, we8, w1, w2, w3, h):
    TB = 512
    IST = 1408
    return pl.pallas_call(
        _moe_body,
        grid=(S // TB, NE, IS // IST),
        in_specs=[
            pl.BlockSpec((TB, DIM), lambda t, e, i: (t, 0)),
            pl.BlockSpec((1, IST, DIM), lambda t, e, i: (e, i, 0)),
            pl.BlockSpec((1, IST, DIM), lambda t, e, i: (e, i, 0)),
            pl.BlockSpec((1, DIM, IST), lambda t, e, i: (e, 0, i)),
            pl.BlockSpec((TB, NE), lambda t, e, i: (t, 0)),
            pl.BlockSpec((TB, DIM), lambda t, e, i: (t, 0)),
        ],
        out_specs=pl.BlockSpec((TB, DIM), lambda t, e, i: (t, 0)),
        out_shape=jax.ShapeDtypeStruct((S, DIM), F32),
        compiler_params=pltpu.CompilerParams(
            dimension_semantics=("parallel", "arbitrary", "arbitrary")),
    )(hn, w1, w3, w2, we8, h)


def _vocab_body(x_ref, nw_ref, ow_ref, o_ref):
    xn = _rms(x_ref[...], nw_ref[...])
    o_ref[...] = _bdot(xn, ow_ref[...], (((1,), (1,)), ((), ())))


def _vocab_call(xf, nw, out_w):
    TB = 256
    VB = 3200
    return pl.pallas_call(
        _vocab_body,
        grid=(S // TB, VOCAB // VB),
        in_specs=[
            pl.BlockSpec((TB, DIM), lambda t, v: (t, 0)),
            pl.BlockSpec((1, DIM), lambda t, v: (0, 0)),
            pl.BlockSpec((VB, DIM), lambda t, v: (v, 0)),
        ],
        out_specs=pl.BlockSpec((TB, VB), lambda t, v: (t, v)),
        out_shape=jax.ShapeDtypeStruct((S, VOCAB), F32),
        compiler_params=pltpu.CompilerParams(
            dimension_semantics=("parallel", "arbitrary")),
    )(xf, nw, out_w)


def kernel(idx, input_pos, tok_emb, wqkv, wo, attn_norm_w, ffn_norm_w,
           gate_w, w1, w2, w3, final_norm_w, out_w):
    idx_flat = idx.reshape(S).astype(jnp.int32)
    cos = jnp.take(jnp.asarray(_COS_NP), input_pos, axis=0)
    sin = jnp.take(jnp.asarray(_SIN_NP), input_pos, axis=0)
    wqkv_p = jnp.take(wqkv, jnp.asarray(_PERM_NP), axis=0)

    x = _embed_gather(tok_emb, idx_flat)
    q3, k3, v3 = _qkv_call(x, attn_norm_w.reshape(1, DIM), wqkv_p, cos, sin)
    attn = _attn_call(q3, k3, v3)
    h, hn = _wo_call(x, attn, wo, ffn_norm_w.reshape(1, DIM))
    we8 = _gate_call(hn, gate_w)
    xf = _moe_call(hn, we8, w1, w2, w3, h)
    logits = _vocab_call(xf, final_norm_w.reshape(1, DIM), out_w)
    return logits.reshape(B, S, VOCAB)
